# Initial kernel scaffold; baseline (speedup 1.0000x reference)
import jax
import jax.numpy as jnp
from jax.experimental import pallas as pl


def kernel(x, table):
    raise NotImplementedError("write your pallas kernel here")


# speedup vs baseline: 1.1113x; 1.1113x over previous
import functools

import jax
import jax.numpy as jnp
from jax import lax
from jax.experimental import pallas as pl
from jax.experimental.pallas import tpu as pltpu
from jax.experimental.pallas import tpu_sc as plsc

_B = 16384 * 50
_D = 32
_NC, _NS = 2, 16
_NW = _NC * _NS
_BPW = _B // _NW
_K = 20
_C = _K * 128
_G = _BPW // _C

_mesh = plsc.VectorSubcoreMesh(core_axis_name="c", subcore_axis_name="s")


@functools.partial(
    pl.kernel,
    mesh=_mesh,
    compiler_params=pltpu.CompilerParams(use_tc_tiling_on_sc=False),
    out_type=jax.ShapeDtypeStruct((_B, _D), jnp.float32),
    scratch_types=[
        pltpu.VMEM((_BPW // 128, 128), jnp.int32),
        pltpu.VMEM((_C, _D), jnp.float32),
        pltpu.SemaphoreType.DMA,
    ],
)
def _embed_lookup(x_hbm, table_hbm, out_hbm, idx_v, rows_v, sem):
    wid = lax.axis_index("s") * _NC + lax.axis_index("c")
    pltpu.sync_copy(x_hbm.at[pl.ds(wid * (_BPW // 128), _BPW // 128)], idx_v)

    def body(g, carry):
        base = wid * _BPW + g * _C
        handles = [
            pltpu.async_copy(
                table_hbm.at[idx_v.at[g * _K + j]],
                rows_v.at[pl.ds(j * 128, 128)],
                sem,
            )
            for j in range(_K)
        ]
        for h in handles:
            h.wait()
        pltpu.sync_copy(rows_v, out_hbm.at[pl.ds(base, _C)])
        return carry

    lax.fori_loop(0, _G, body, None)


def kernel(x, table):
    x2d = x.reshape(_B // 128, 128)
    out = _embed_lookup(x2d, table)
    return out.reshape(x.shape + (_D,))


# speedup vs baseline: 1.1129x; 1.0014x over previous
import functools

import jax
import jax.numpy as jnp
from jax import lax
from jax.experimental import pallas as pl
from jax.experimental.pallas import tpu as pltpu
from jax.experimental.pallas import tpu_sc as plsc

_B = 16384 * 50
_D = 32
_NC, _NS = 2, 16
_NW = _NC * _NS
_BPW = _B // _NW
_IR = _BPW // 128
_K = 10
_C = _K * 128
_G = _BPW // _C
_PAIRS = _G // 2

_mesh = plsc.VectorSubcoreMesh(core_axis_name="c", subcore_axis_name="s")


@functools.partial(
    pl.kernel,
    mesh=_mesh,
    compiler_params=pltpu.CompilerParams(use_tc_tiling_on_sc=False),
    out_type=jax.ShapeDtypeStruct((_B, _D), jnp.float32),
    scratch_types=[
        pltpu.VMEM((_IR, 128), jnp.int32),
        pltpu.VMEM((_C, _D), jnp.float32),
        pltpu.VMEM((_C, _D), jnp.float32),
        pltpu.SemaphoreType.DMA,
        pltpu.SemaphoreType.DMA,
        pltpu.SemaphoreType.DMA,
        pltpu.SemaphoreType.DMA,
    ],
)
def _embed_lookup(x_hbm, table_hbm, out_hbm, idx_v, rows0, rows1,
                  g0, g1, s0, s1):
    wid = lax.axis_index("s") * _NC + lax.axis_index("c")
    base = wid * _BPW

    def fire_gathers(rows, sem, chunk):
        for j in range(_K):
            pltpu.make_async_copy(
                table_hbm.at[idx_v.at[chunk * _K + j]],
                rows.at[pl.ds(j * 128, 128)],
                sem,
            ).start()

    def drain_gathers(rows, sem, chunk):
        for j in range(_K):
            pltpu.make_async_copy(
                table_hbm.at[idx_v.at[chunk * _K + j]],
                rows.at[pl.ds(j * 128, 128)],
                sem,
            ).wait()

    def store(rows, sem, chunk):
        return pltpu.make_async_copy(
            rows, out_hbm.at[pl.ds(base + chunk * _C, _C)], sem)

    pltpu.sync_copy(x_hbm.at[pl.ds(wid * _IR, _IR)], idx_v)
    fire_gathers(rows0, g0, 0)

    def body(t, carry):
        a = 2 * t
        b = 2 * t + 1

        @pl.when(t > 0)
        def _():
            store(rows1, s1, b - 2).wait()
        fire_gathers(rows1, g1, b)

        drain_gathers(rows0, g0, a)
        store(rows0, s0, a).start()

        @pl.when(t < _PAIRS - 1)
        def _():
            store(rows0, s0, a).wait()
            fire_gathers(rows0, g0, a + 2)

        drain_gathers(rows1, g1, b)
        store(rows1, s1, b).start()
        return carry

    lax.fori_loop(0, _PAIRS, body, None)
    store(rows0, s0, _G - 2).wait()
    store(rows1, s1, _G - 1).wait()


def kernel(x, table):
    x2d = x.reshape(_B // 128, 128)
    out = _embed_lookup(x2d, table)
    return out.reshape(x.shape + (_D,))


# speedup vs baseline: 1.1142x; 1.0012x over previous
import functools

import jax
import jax.numpy as jnp
from jax import lax
from jax.experimental import pallas as pl
from jax.experimental.pallas import tpu as pltpu
from jax.experimental.pallas import tpu_sc as plsc

_ROWS = 16384
_S = 50
_D = 32
_B = _ROWS * _S
_NC, _NS = 2, 16
_NW = _NC * _NS
_BPW = _B // _NW
_IR = _BPW // 128
_K = 10
_C = _K * 128
_G = _BPW // _C
_PAIRS = _G // 2

_mesh = plsc.VectorSubcoreMesh(core_axis_name="c", subcore_axis_name="s")


@functools.partial(
    pl.kernel,
    mesh=_mesh,
    compiler_params=pltpu.CompilerParams(use_tc_tiling_on_sc=False),
    out_type=jax.ShapeDtypeStruct((_B, _D), jnp.float32),
    scratch_types=[
        pltpu.VMEM((_IR, 128), jnp.int32),
        pltpu.VMEM((_C, _D), jnp.float32),
        pltpu.VMEM((_C, _D), jnp.float32),
        pltpu.SemaphoreType.DMA,
        pltpu.SemaphoreType.DMA,
        pltpu.SemaphoreType.DMA,
        pltpu.SemaphoreType.DMA,
    ],
)
def _embed_lookup(x_hbm, table_hbm, out_hbm, flat_v, rows0, rows1,
                  g0, g1, s0, s1):
    wid = lax.axis_index("s") * _NC + lax.axis_index("c")
    base = wid * _BPW

    pltpu.sync_copy(x_hbm.at[pl.ds(wid * _IR, _IR)], flat_v)

    def fire_gathers(rows, sem, chunk):
        for k in range(_K):
            pltpu.make_async_copy(
                table_hbm.at[flat_v.at[chunk * _K + k]],
                rows.at[pl.ds(k * 128, 128)],
                sem,
            ).start()

    def drain_gathers(rows, sem, chunk):
        for k in range(_K):
            pltpu.make_async_copy(
                table_hbm.at[flat_v.at[chunk * _K + k]],
                rows.at[pl.ds(k * 128, 128)],
                sem,
            ).wait()

    def store(rows, sem, chunk):
        return pltpu.make_async_copy(
            rows,
            out_hbm.at[pl.ds(base + chunk * _C, _C)],
            sem,
        )

    fire_gathers(rows0, g0, 0)

    def body(t, carry):
        a = 2 * t
        b = 2 * t + 1

        @pl.when(t > 0)
        def _():
            store(rows1, s1, b - 2).wait()
        fire_gathers(rows1, g1, b)

        drain_gathers(rows0, g0, a)
        store(rows0, s0, a).start()

        @pl.when(t < _PAIRS - 1)
        def _():
            store(rows0, s0, a).wait()
            fire_gathers(rows0, g0, a + 2)

        drain_gathers(rows1, g1, b)
        store(rows1, s1, b).start()
        return carry

    lax.fori_loop(0, _PAIRS, body, None)
    store(rows0, s0, _G - 2).wait()
    store(rows1, s1, _G - 1).wait()


def kernel(x, table):
    x_flat = x.reshape(_B // 128, 128)
    out2 = _embed_lookup(x_flat, table)
    return out2.reshape(_ROWS, _S, _D)


# speedup vs baseline: 1.3111x; 1.1767x over previous
import functools

import jax
import jax.numpy as jnp
from jax import lax
from jax.experimental import pallas as pl
from jax.experimental.pallas import tpu as pltpu
from jax.experimental.pallas import tpu_sc as plsc

_ROWS = 16384
_S = 50
_D = 32
_B = _ROWS * _S
_NC, _NS = 2, 16
_NW = _NC * _NS
_BPW = _B // _NW
_IR = _BPW // 128
_K = 10
_C = _K * 128
_G = _BPW // _C
_PAIRS = _G // 2

_mesh = plsc.VectorSubcoreMesh(core_axis_name="c", subcore_axis_name="s")


@functools.partial(
    pl.kernel,
    mesh=_mesh,
    compiler_params=pltpu.CompilerParams(use_tc_tiling_on_sc=False),
    out_type=jax.ShapeDtypeStruct((_B // 128, 128, _D), jnp.float32),
    scratch_types=[
        pltpu.VMEM((_IR, 128), jnp.int32),
        pltpu.VMEM((_K, 128, _D), jnp.float32),
        pltpu.VMEM((_K, 128, _D), jnp.float32),
        pltpu.SemaphoreType.DMA,
        pltpu.SemaphoreType.DMA,
        pltpu.SemaphoreType.DMA,
        pltpu.SemaphoreType.DMA,
    ],
)
def _embed_lookup(x_hbm, table_hbm, out_hbm, flat_v, rows0, rows1,
                  g0, g1, s0, s1):
    wid = lax.axis_index("s") * _NC + lax.axis_index("c")
    base = wid * _IR

    pltpu.sync_copy(x_hbm.at[pl.ds(base, _IR)], flat_v)

    def gather_k(rows, sem, chunk, k):
        return pltpu.make_async_copy(
            table_hbm.at[flat_v.at[chunk * _K + k]],
            rows.at[k],
            sem,
        )

    class _Chunk:
        def __init__(self, rows, sem, chunk):
            self.rows, self.sem, self.chunk = rows, sem, chunk

        def start(self):
            for k in range(_K):
                gather_k(self.rows, self.sem, self.chunk, k).start()

        def wait(self):
            for k in range(_K):
                gather_k(self.rows, self.sem, self.chunk, k).wait()

    def gather(rows, sem, chunk):
        return _Chunk(rows, sem, chunk)

    def store(rows, sem, chunk):
        return pltpu.make_async_copy(
            rows,
            out_hbm.at[pl.ds(base + chunk * _K, _K)],
            sem,
        )

    gather(rows0, g0, 0).start()

    def body(t, carry):
        a = 2 * t
        b = 2 * t + 1

        @pl.when(t > 0)
        def _():
            store(rows1, s1, b - 2).wait()
        gather(rows1, g1, b).start()

        gather(rows0, g0, a).wait()
        store(rows0, s0, a).start()

        @pl.when(t < _PAIRS - 1)
        def _():
            store(rows0, s0, a).wait()
            gather(rows0, g0, a + 2).start()

        gather(rows1, g1, b).wait()
        store(rows1, s1, b).start()
        return carry

    lax.fori_loop(0, _PAIRS, body, None)
    store(rows0, s0, _G - 2).wait()
    store(rows1, s1, _G - 1).wait()


def kernel(x, table):
    x_flat = x.reshape(_B // 128, 128)
    out3 = _embed_lookup(x_flat, table)
    return out3.reshape(_ROWS, _S, _D)


# speedup vs baseline: 1.9426x; 1.4816x over previous
import functools

import jax
import jax.numpy as jnp
from jax import lax
from jax.experimental import pallas as pl
from jax.experimental.pallas import tpu as pltpu
from jax.experimental.pallas import tpu_sc as plsc

_ROWS = 16384
_S = 50
_D = 32
_B = _ROWS * _S
_NC, _NS = 2, 16
_NW = _NC * _NS
_BPW = _B // _NW
_IR = _BPW // 128
_K = 10
_C = _K * 128
_G = _BPW // _C
_PAIRS = _G // 2

_mesh = plsc.VectorSubcoreMesh(core_axis_name="c", subcore_axis_name="s")


@functools.partial(
    pl.kernel,
    mesh=_mesh,
    compiler_params=pltpu.CompilerParams(use_tc_tiling_on_sc=False),
    out_type=jax.ShapeDtypeStruct((_B // 128, 128, _D), jnp.float32),
    scratch_types=[
        pltpu.VMEM((_IR, 128), jnp.int32),
        pltpu.VMEM((_K, 128, _D), jnp.float32),
        pltpu.VMEM((_K, 128, _D), jnp.float32),
        pltpu.SemaphoreType.DMA,
        pltpu.SemaphoreType.DMA,
        pltpu.SemaphoreType.DMA,
        pltpu.SemaphoreType.DMA,
    ],
)
def _embed_lookup(x_hbm, table_hbm, out_hbm, flat_v, rows0, rows1,
                  g0, g1, s0, s1):
    wid = lax.axis_index("s") * _NC + lax.axis_index("c")
    base = wid * _IR

    pltpu.sync_copy(x_hbm.at[pl.ds(base, _IR)], flat_v)

    def gather_k(rows, sem, chunk, k):
        return pltpu.make_async_copy(
            table_hbm.at[flat_v.at[chunk * _K + k]],
            rows.at[k],
            sem,
        )

    class _Chunk:
        def __init__(self, rows, sem, chunk):
            self.rows, self.sem, self.chunk = rows, sem, chunk

        def start(self):
            for k in range(_K):
                gather_k(self.rows, self.sem, self.chunk, k).start()

        def wait(self):
            for k in range(_K):
                gather_k(self.rows, self.sem, self.chunk, k).wait()

    def gather(rows, sem, chunk):
        return _Chunk(rows, sem, chunk)

    def store(rows, sem, chunk):
        return pltpu.make_async_copy(
            rows,
            out_hbm.at[pl.ds(base + chunk * _K, _K)],
            sem,
        )

    gather(rows0, g0, 0).start()

    def body(t, carry):
        a = 2 * t
        b = 2 * t + 1

        @pl.when(t > 0)
        def _():
            store(rows1, s1, b - 2).wait()
        gather(rows1, g1, b).start()

        gather(rows0, g0, a).wait()
        store(rows0, s0, a).start()

        @pl.when(t < _PAIRS - 1)
        def _():
            store(rows0, s0, a).wait()
            gather(rows0, g0, a + 2).start()

        gather(rows1, g1, b).wait()
        store(rows1, s1, b).start()
        return carry

    lax.fori_loop(0, _PAIRS, body, None)
    store(rows0, s0, _G - 2).wait()
    store(rows1, s1, _G - 1).wait()


def kernel(x, table):
    x_flat = x.T.reshape(_B // 128, 128)
    out3 = _embed_lookup(x_flat, table)
    return out3.reshape(_S, _ROWS, _D).transpose(1, 0, 2)
